# baseline (speedup 1.0000x reference)
import functools
import numpy as np
import jax
import jax.numpy as jnp
from jax import lax
from jax.experimental import pallas as pl
from jax.experimental.pallas import tpu as pltpu
from jax.experimental.pallas import tpu_sc as plsc

_B = 4096
_F = 26
_V = 100000
_D = 64
_NF = _F + 1
_NUM_INTER = (_NF * (_NF - 1)) // 2
_B_BLK = 256
_NW = 32
_CH = 128
_VB = 1024
_NVB = 49
_HALF = _VB * _NVB


def _conv_body(a_ref, b_ref, eye_ref, out_ref):
  c00 = (((0,), (0,)), ((), ()))
  eye = eye_ref[...]
  ta = lax.dot_general(a_ref[...], eye, c00, preferred_element_type=jnp.float32)
  tb = lax.dot_general(b_ref[...], eye, c00, preferred_element_type=jnp.float32)
  out_ref[...] = jnp.concatenate([ta, tb], axis=1)


def _tc_convert(table2):
  return pl.pallas_call(
      _conv_body,
      grid=(_F, _NVB),
      in_specs=[
          pl.BlockSpec((_D, _VB), lambda f, v: (f, v)),
          pl.BlockSpec((_D, _VB), lambda f, v: (f, _NVB + v)),
          pl.BlockSpec((_D, _D), lambda f, v: (0, 0)),
      ],
      out_specs=pl.BlockSpec((_VB, 2 * _D), lambda f, v: (f * _NVB + v, 0)),
      out_shape=jax.ShapeDtypeStruct((_F * _HALF, 2 * _D), jnp.float32),
  )(table2, table2, jnp.eye(_D, dtype=jnp.float32))


def _sc_gather(table, pair_idx):
  R = pair_idx.shape[0]
  r_per_w = R // _NW
  n_ch = r_per_w // _CH
  mesh = plsc.VectorSubcoreMesh(core_axis_name="c", subcore_axis_name="s")

  @functools.partial(
      pl.kernel,
      mesh=mesh,
      out_type=jax.ShapeDtypeStruct((R, 2 * _D), jnp.float32),
      scratch_types=[
          pltpu.VMEM((_CH,), jnp.int32),
          pltpu.VMEM((_CH,), jnp.int32),
          pltpu.VMEM((_CH, 2 * _D), jnp.float32),
          pltpu.VMEM((_CH, 2 * _D), jnp.float32),
          pltpu.SemaphoreType.DMA,
          pltpu.SemaphoreType.DMA,
          pltpu.SemaphoreType.DMA,
      ],
  )
  def gather_k(table_hbm, idx_hbm, out_hbm, idx0, idx1, rows0, rows1,
               isem, gsem, osem):
    wid = lax.axis_index("s") * 2 + lax.axis_index("c")
    base = wid * r_per_w
    idx = (idx0, idx1)
    rows = (rows0, rows1)

    ic = [None, None]
    oc = [None, None]
    ic[0] = pltpu.async_copy(idx_hbm.at[pl.ds(base, _CH)], idx0, isem)
    for c in range(n_ch):
      p = c % 2
      b0 = base + c * _CH
      ic[p].wait()
      if oc[p] is not None:
        oc[p].wait()
      g = pltpu.async_copy(table_hbm.at[idx[p]], rows[p], gsem)
      if c + 1 < n_ch:
        ic[1 - p] = pltpu.async_copy(
            idx_hbm.at[pl.ds(b0 + _CH, _CH)], idx[1 - p], isem)
      g.wait()
      oc[p] = pltpu.async_copy(rows[p], out_hbm.at[pl.ds(b0, _CH)], osem)
    for cp in oc:
      if cp is not None:
        cp.wait()

  return gather_k(table, pair_idx)


def _tc_body(emb_ref, par_ref, xdt_ref, bw0, bb0, bw1, bb1, bw2, bb2,
             tw0p, tb0, tw1, tb1, tw2, tb2, out_ref, et_ref, tt_ref):
  c00 = (((0,), (0,)), ((), ()))
  f32 = jnp.float32
  h = jnp.maximum(
      lax.dot_general(bw0[...], xdt_ref[...], c00, preferred_element_type=f32)
      + bb0[...], 0.0)
  h = jnp.maximum(
      lax.dot_general(bw1[...], h, c00, preferred_element_type=f32)
      + bb1[...], 0.0)
  dot_t = (lax.dot_general(bw2[...], h, c00, preferred_element_type=f32)
           + bb2[...])

  for f in range(_F):
    mt = jnp.transpose(emb_ref[f])
    odd = par_ref[f].reshape(1, _B_BLK) == 1
    et_ref[f * _D:(f + 1) * _D, :] = jnp.where(
        odd, mt[_D:2 * _D, :], mt[:_D, :])

  tt_ref[_NUM_INTER + 1:_NUM_INTER + 1 + _D, :] = dot_t
  tt_ref[_NUM_INTER:_NUM_INTER + 1, :] = jnp.zeros((1, _B_BLK), f32)

  p = 0
  for i in range(_NF):
    ai = et_ref[i * _D:(i + 1) * _D, :] if i < _F else dot_t
    for j in range(i + 1, _NF):
      aj = et_ref[j * _D:(j + 1) * _D, :] if j < _F else dot_t
      s = jnp.sum(ai * aj, axis=0, keepdims=True)
      tt_ref[p:p + 1, :] = s
      p += 1

  tt = tt_ref[...]
  h2 = jnp.maximum(
      lax.dot_general(tt, tw0p[...], c00, preferred_element_type=f32)
      + tb0[...], 0.0)
  h3 = jnp.maximum(
      jnp.dot(h2, tw1[...], preferred_element_type=f32) + tb1[...], 0.0)
  out_ref[...] = jnp.dot(h3, tw2[...], preferred_element_type=f32) + tb2[...]


def _tc_dense(emb3, par, xdt, bw0, bb0, bw1, bb1, bw2, bb2,
              tw0p, tb0, tw1, tb1, tw2, tb2):
  n_blk = _B // _B_BLK
  full = lambda shape: pl.BlockSpec(shape, lambda i: (0,) * len(shape))
  return pl.pallas_call(
      _tc_body,
      grid=(n_blk,),
      in_specs=[
          pl.BlockSpec((_F, _B_BLK, 2 * _D), lambda i: (0, i, 0)),
          pl.BlockSpec((_F, _B_BLK), lambda i: (0, i)),
          pl.BlockSpec((13, _B_BLK), lambda i: (0, i)),
          full(bw0.shape), full(bb0.shape), full(bw1.shape), full(bb1.shape),
          full(bw2.shape), full(bb2.shape), full(tw0p.shape), full(tb0.shape),
          full(tw1.shape), full(tb1.shape), full(tw2.shape), full(tb2.shape),
      ],
      out_specs=pl.BlockSpec((_B_BLK, 1), lambda i: (i, 0)),
      out_shape=jax.ShapeDtypeStruct((_B, 1), jnp.float32),
      scratch_shapes=[
          pltpu.VMEM((_F * _D, _B_BLK), jnp.float32),
          pltpu.VMEM((_NUM_INTER + 1 + _D, _B_BLK), jnp.float32),
      ],
  )(emb3, par, xdt, bw0, bb0, bw1, bb1, bw2, bb2,
    tw0p, tb0, tw1, tb1, tw2, tb2)


def kernel(x_sparse, x_dense, emb, bw0, bb0, bw1, bb1, bw2, bb2,
           tw0, tb0, tw1, tb1, tw2, tb2):
  table2 = jnp.transpose(emb, (0, 2, 1)).reshape(_F * _D, _V)
  table = _tc_convert(table2)

  xs_t = x_sparse.T
  par = (xs_t >= _HALF).astype(jnp.int32)
  pair_idx = (jnp.arange(_F, dtype=jnp.int32)[:, None] * _HALF
              + xs_t - par * _HALF).reshape(-1)
  embeds = _sc_gather(table, pair_idx)
  emb3 = embeds.reshape(_F, _B, 2 * _D)

  tw0p = jnp.concatenate(
      [tw0[:_NUM_INTER], jnp.zeros((1, tw0.shape[1]), tw0.dtype),
       tw0[_NUM_INTER:]], axis=0)
  out = _tc_dense(
      emb3, par, x_dense.T,
      bw0, bb0.reshape(-1, 1), bw1, bb1.reshape(-1, 1), bw2, bb2.reshape(-1, 1),
      tw0p, tb0.reshape(1, -1), tw1, tb1.reshape(1, -1), tw2, tb2.reshape(1, -1))
  return out


# speedup vs baseline: 1.8164x; 1.8164x over previous
import functools
import numpy as np
import jax
import jax.numpy as jnp
from jax import lax
from jax.experimental import pallas as pl
from jax.experimental.pallas import tpu as pltpu
from jax.experimental.pallas import tpu_sc as plsc

_B = 4096
_F = 26
_V = 100000
_D = 64
_NF = _F + 1
_NUM_INTER = (_NF * (_NF - 1)) // 2
_B_BLK = 256
_NW = 32
_CH = 128
_VB = 6272
_NVB = 8
_HALF = _VB * _NVB


def _conv_body(a_ref, b_ref, eye_ref, out_ref):
  c00 = (((0,), (0,)), ((), ()))
  eye = eye_ref[...]
  ta = lax.dot_general(a_ref[...], eye, c00, preferred_element_type=jnp.float32)
  tb = lax.dot_general(b_ref[...], eye, c00, preferred_element_type=jnp.float32)
  out_ref[...] = jnp.concatenate([ta, tb], axis=1)


def _tc_convert(table2):
  return pl.pallas_call(
      _conv_body,
      grid=(_F, _NVB),
      in_specs=[
          pl.BlockSpec((_D, _VB), lambda f, v: (f, v)),
          pl.BlockSpec((_D, _VB), lambda f, v: (f, _NVB + v)),
          pl.BlockSpec((_D, _D), lambda f, v: (0, 0)),
      ],
      out_specs=pl.BlockSpec((_VB, 2 * _D), lambda f, v: (f * _NVB + v, 0)),
      out_shape=jax.ShapeDtypeStruct((_F * _HALF, 2 * _D), jnp.float32),
  )(table2, table2, jnp.eye(_D, dtype=jnp.float32))


def _sc_gather(table, pair_idx):
  R = pair_idx.shape[0]
  r_per_w = R // _NW
  n_ch = r_per_w // _CH
  mesh = plsc.VectorSubcoreMesh(core_axis_name="c", subcore_axis_name="s")

  @functools.partial(
      pl.kernel,
      mesh=mesh,
      out_type=jax.ShapeDtypeStruct((R, 2 * _D), jnp.float32),
      scratch_types=[
          pltpu.VMEM((_CH,), jnp.int32),
          pltpu.VMEM((_CH,), jnp.int32),
          pltpu.VMEM((_CH, 2 * _D), jnp.float32),
          pltpu.VMEM((_CH, 2 * _D), jnp.float32),
          pltpu.SemaphoreType.DMA,
          pltpu.SemaphoreType.DMA,
          pltpu.SemaphoreType.DMA,
      ],
  )
  def gather_k(table_hbm, idx_hbm, out_hbm, idx0, idx1, rows0, rows1,
               isem, gsem, osem):
    wid = lax.axis_index("s") * 2 + lax.axis_index("c")
    base = wid * r_per_w
    idx = (idx0, idx1)
    rows = (rows0, rows1)

    ic = [None, None]
    oc = [None, None]
    ic[0] = pltpu.async_copy(idx_hbm.at[pl.ds(base, _CH)], idx0, isem)
    for c in range(n_ch):
      p = c % 2
      b0 = base + c * _CH
      ic[p].wait()
      if oc[p] is not None:
        oc[p].wait()
      g = pltpu.async_copy(table_hbm.at[idx[p]], rows[p], gsem)
      if c + 1 < n_ch:
        ic[1 - p] = pltpu.async_copy(
            idx_hbm.at[pl.ds(b0 + _CH, _CH)], idx[1 - p], isem)
      g.wait()
      oc[p] = pltpu.async_copy(rows[p], out_hbm.at[pl.ds(b0, _CH)], osem)
    for cp in oc:
      if cp is not None:
        cp.wait()

  return gather_k(table, pair_idx)


def _tc_body(emb_ref, par_ref, xdt_ref, bw0, bb0, bw1, bb1, bw2, bb2,
             tw0p, tb0, tw1, tb1, tw2, tb2, out_ref, et_ref, tt_ref):
  c00 = (((0,), (0,)), ((), ()))
  f32 = jnp.float32
  h = jnp.maximum(
      lax.dot_general(bw0[...], xdt_ref[...], c00, preferred_element_type=f32)
      + bb0[...], 0.0)
  h = jnp.maximum(
      lax.dot_general(bw1[...], h, c00, preferred_element_type=f32)
      + bb1[...], 0.0)
  dot_t = (lax.dot_general(bw2[...], h, c00, preferred_element_type=f32)
           + bb2[...])

  for f in range(_F):
    mt = jnp.transpose(emb_ref[f])
    odd = par_ref[f].reshape(1, _B_BLK) == 1
    et_ref[f * _D:(f + 1) * _D, :] = jnp.where(
        odd, mt[_D:2 * _D, :], mt[:_D, :])

  tt_ref[_NUM_INTER + 1:_NUM_INTER + 1 + _D, :] = dot_t
  tt_ref[_NUM_INTER:_NUM_INTER + 1, :] = jnp.zeros((1, _B_BLK), f32)

  p = 0
  for i in range(_NF):
    ai = et_ref[i * _D:(i + 1) * _D, :] if i < _F else dot_t
    for j in range(i + 1, _NF):
      aj = et_ref[j * _D:(j + 1) * _D, :] if j < _F else dot_t
      s = jnp.sum(ai * aj, axis=0, keepdims=True)
      tt_ref[p:p + 1, :] = s
      p += 1

  tt = tt_ref[...]
  h2 = jnp.maximum(
      lax.dot_general(tt, tw0p[...], c00, preferred_element_type=f32)
      + tb0[...], 0.0)
  h3 = jnp.maximum(
      jnp.dot(h2, tw1[...], preferred_element_type=f32) + tb1[...], 0.0)
  out_ref[...] = jnp.dot(h3, tw2[...], preferred_element_type=f32) + tb2[...]


def _tc_dense(emb3, par, xdt, bw0, bb0, bw1, bb1, bw2, bb2,
              tw0p, tb0, tw1, tb1, tw2, tb2):
  n_blk = _B // _B_BLK
  full = lambda shape: pl.BlockSpec(shape, lambda i: (0,) * len(shape))
  return pl.pallas_call(
      _tc_body,
      grid=(n_blk,),
      in_specs=[
          pl.BlockSpec((_F, _B_BLK, 2 * _D), lambda i: (0, i, 0)),
          pl.BlockSpec((_F, _B_BLK), lambda i: (0, i)),
          pl.BlockSpec((13, _B_BLK), lambda i: (0, i)),
          full(bw0.shape), full(bb0.shape), full(bw1.shape), full(bb1.shape),
          full(bw2.shape), full(bb2.shape), full(tw0p.shape), full(tb0.shape),
          full(tw1.shape), full(tb1.shape), full(tw2.shape), full(tb2.shape),
      ],
      out_specs=pl.BlockSpec((_B_BLK, 1), lambda i: (i, 0)),
      out_shape=jax.ShapeDtypeStruct((_B, 1), jnp.float32),
      scratch_shapes=[
          pltpu.VMEM((_F * _D, _B_BLK), jnp.float32),
          pltpu.VMEM((_NUM_INTER + 1 + _D, _B_BLK), jnp.float32),
      ],
  )(emb3, par, xdt, bw0, bb0, bw1, bb1, bw2, bb2,
    tw0p, tb0, tw1, tb1, tw2, tb2)


def kernel(x_sparse, x_dense, emb, bw0, bb0, bw1, bb1, bw2, bb2,
           tw0, tb0, tw1, tb1, tw2, tb2):
  table2 = jnp.transpose(emb, (0, 2, 1)).reshape(_F * _D, _V)
  table = _tc_convert(table2)

  xs_t = x_sparse.T
  par = (xs_t >= _HALF).astype(jnp.int32)
  pair_idx = (jnp.arange(_F, dtype=jnp.int32)[:, None] * _HALF
              + xs_t - par * _HALF).reshape(-1)
  embeds = _sc_gather(table, pair_idx)
  emb3 = embeds.reshape(_F, _B, 2 * _D)

  tw0p = jnp.concatenate(
      [tw0[:_NUM_INTER], jnp.zeros((1, tw0.shape[1]), tw0.dtype),
       tw0[_NUM_INTER:]], axis=0)
  out = _tc_dense(
      emb3, par, x_dense.T,
      bw0, bb0.reshape(-1, 1), bw1, bb1.reshape(-1, 1), bw2, bb2.reshape(-1, 1),
      tw0p, tb0.reshape(1, -1), tw1, tb1.reshape(1, -1), tw2, tb2.reshape(1, -1))
  return out


# speedup vs baseline: 1.9805x; 1.0903x over previous
import functools
import numpy as np
import jax
import jax.numpy as jnp
from jax import lax
from jax.experimental import pallas as pl
from jax.experimental.pallas import tpu as pltpu
from jax.experimental.pallas import tpu_sc as plsc

_B = 4096
_F = 26
_V = 100000
_D = 64
_NF = _F + 1
_NUM_INTER = (_NF * (_NF - 1)) // 2
_B_BLK = 256
_NW = 32
_CH = 128
_VB = 12544
_NVB = 4
_HALF = _VB * _NVB


def _conv_body(a_ref, b_ref, eye_ref, out_ref):
  c00 = (((0,), (0,)), ((), ()))
  eye = eye_ref[...]
  ta = lax.dot_general(a_ref[...], eye, c00, preferred_element_type=jnp.float32)
  tb = lax.dot_general(b_ref[...], eye, c00, preferred_element_type=jnp.float32)
  out_ref[...] = jnp.concatenate([ta, tb], axis=1)


def _tc_convert(table2):
  return pl.pallas_call(
      _conv_body,
      grid=(_F, _NVB),
      in_specs=[
          pl.BlockSpec((_D, _VB), lambda f, v: (f, v)),
          pl.BlockSpec((_D, _VB), lambda f, v: (f, _NVB + v)),
          pl.BlockSpec((_D, _D), lambda f, v: (0, 0)),
      ],
      out_specs=pl.BlockSpec((_VB, 2 * _D), lambda f, v: (f * _NVB + v, 0)),
      out_shape=jax.ShapeDtypeStruct((_F * _HALF, 2 * _D), jnp.float32),
  )(table2, table2, jnp.eye(_D, dtype=jnp.float32))


def _sc_gather(table, pair_idx):
  R = pair_idx.shape[0]
  r_per_w = R // _NW
  n_ch = r_per_w // _CH
  mesh = plsc.VectorSubcoreMesh(core_axis_name="c", subcore_axis_name="s")

  @functools.partial(
      pl.kernel,
      mesh=mesh,
      out_type=jax.ShapeDtypeStruct((R, 2 * _D), jnp.float32),
      scratch_types=[
          pltpu.VMEM((_CH,), jnp.int32),
          pltpu.VMEM((_CH,), jnp.int32),
          pltpu.VMEM((_CH, 2 * _D), jnp.float32),
          pltpu.VMEM((_CH, 2 * _D), jnp.float32),
          pltpu.SemaphoreType.DMA,
          pltpu.SemaphoreType.DMA,
          pltpu.SemaphoreType.DMA,
      ],
  )
  def gather_k(table_hbm, idx_hbm, out_hbm, idx0, idx1, rows0, rows1,
               isem, gsem, osem):
    wid = lax.axis_index("s") * 2 + lax.axis_index("c")
    base = wid * r_per_w
    idx = (idx0, idx1)
    rows = (rows0, rows1)

    ic = [None, None]
    oc = [None, None]
    ic[0] = pltpu.async_copy(idx_hbm.at[pl.ds(base, _CH)], idx0, isem)
    for c in range(n_ch):
      p = c % 2
      b0 = base + c * _CH
      ic[p].wait()
      if oc[p] is not None:
        oc[p].wait()
      g = pltpu.async_copy(table_hbm.at[idx[p]], rows[p], gsem)
      if c + 1 < n_ch:
        ic[1 - p] = pltpu.async_copy(
            idx_hbm.at[pl.ds(b0 + _CH, _CH)], idx[1 - p], isem)
      g.wait()
      oc[p] = pltpu.async_copy(rows[p], out_hbm.at[pl.ds(b0, _CH)], osem)
    for cp in oc:
      if cp is not None:
        cp.wait()

  return gather_k(table, pair_idx)


def _tc_body(emb_ref, par_ref, xdt_ref, bw0, bb0, bw1, bb1, bw2, bb2,
             tw0p, tb0, tw1, tb1, tw2, tb2, out_ref, et_ref, tt_ref):
  c00 = (((0,), (0,)), ((), ()))
  f32 = jnp.float32
  h = jnp.maximum(
      lax.dot_general(bw0[...], xdt_ref[...], c00, preferred_element_type=f32)
      + bb0[...], 0.0)
  h = jnp.maximum(
      lax.dot_general(bw1[...], h, c00, preferred_element_type=f32)
      + bb1[...], 0.0)
  dot_t = (lax.dot_general(bw2[...], h, c00, preferred_element_type=f32)
           + bb2[...])

  for f in range(_F):
    mt = jnp.transpose(emb_ref[f])
    odd = par_ref[f].reshape(1, _B_BLK) == 1
    et_ref[f * _D:(f + 1) * _D, :] = jnp.where(
        odd, mt[_D:2 * _D, :], mt[:_D, :])

  tt_ref[_NUM_INTER + 1:_NUM_INTER + 1 + _D, :] = dot_t
  tt_ref[_NUM_INTER:_NUM_INTER + 1, :] = jnp.zeros((1, _B_BLK), f32)

  p = 0
  for i in range(_NF):
    ai = et_ref[i * _D:(i + 1) * _D, :] if i < _F else dot_t
    for j in range(i + 1, _NF):
      aj = et_ref[j * _D:(j + 1) * _D, :] if j < _F else dot_t
      s = jnp.sum(ai * aj, axis=0, keepdims=True)
      tt_ref[p:p + 1, :] = s
      p += 1

  tt = tt_ref[...]
  h2 = jnp.maximum(
      lax.dot_general(tt, tw0p[...], c00, preferred_element_type=f32)
      + tb0[...], 0.0)
  h3 = jnp.maximum(
      jnp.dot(h2, tw1[...], preferred_element_type=f32) + tb1[...], 0.0)
  out_ref[...] = jnp.dot(h3, tw2[...], preferred_element_type=f32) + tb2[...]


def _tc_dense(emb3, par, xdt, bw0, bb0, bw1, bb1, bw2, bb2,
              tw0p, tb0, tw1, tb1, tw2, tb2):
  n_blk = _B // _B_BLK
  full = lambda shape: pl.BlockSpec(shape, lambda i: (0,) * len(shape))
  return pl.pallas_call(
      _tc_body,
      grid=(n_blk,),
      in_specs=[
          pl.BlockSpec((_F, _B_BLK, 2 * _D), lambda i: (0, i, 0)),
          pl.BlockSpec((_F, _B_BLK), lambda i: (0, i)),
          pl.BlockSpec((13, _B_BLK), lambda i: (0, i)),
          full(bw0.shape), full(bb0.shape), full(bw1.shape), full(bb1.shape),
          full(bw2.shape), full(bb2.shape), full(tw0p.shape), full(tb0.shape),
          full(tw1.shape), full(tb1.shape), full(tw2.shape), full(tb2.shape),
      ],
      out_specs=pl.BlockSpec((_B_BLK, 1), lambda i: (i, 0)),
      out_shape=jax.ShapeDtypeStruct((_B, 1), jnp.float32),
      scratch_shapes=[
          pltpu.VMEM((_F * _D, _B_BLK), jnp.float32),
          pltpu.VMEM((_NUM_INTER + 1 + _D, _B_BLK), jnp.float32),
      ],
  )(emb3, par, xdt, bw0, bb0, bw1, bb1, bw2, bb2,
    tw0p, tb0, tw1, tb1, tw2, tb2)


def kernel(x_sparse, x_dense, emb, bw0, bb0, bw1, bb1, bw2, bb2,
           tw0, tb0, tw1, tb1, tw2, tb2):
  table2 = jnp.transpose(emb, (0, 2, 1)).reshape(_F * _D, _V)
  table = _tc_convert(table2)

  xs_t = x_sparse.T
  par = (xs_t >= _HALF).astype(jnp.int32)
  pair_idx = (jnp.arange(_F, dtype=jnp.int32)[:, None] * _HALF
              + xs_t - par * _HALF).reshape(-1)
  embeds = _sc_gather(table, pair_idx)
  emb3 = embeds.reshape(_F, _B, 2 * _D)

  tw0p = jnp.concatenate(
      [tw0[:_NUM_INTER], jnp.zeros((1, tw0.shape[1]), tw0.dtype),
       tw0[_NUM_INTER:]], axis=0)
  out = _tc_dense(
      emb3, par, x_dense.T,
      bw0, bb0.reshape(-1, 1), bw1, bb1.reshape(-1, 1), bw2, bb2.reshape(-1, 1),
      tw0p, tb0.reshape(1, -1), tw1, tb1.reshape(1, -1), tw2, tb2.reshape(1, -1))
  return out


# speedup vs baseline: 2.1254x; 1.0732x over previous
import functools
import numpy as np
import jax
import jax.numpy as jnp
from jax import lax
from jax.experimental import pallas as pl
from jax.experimental.pallas import tpu as pltpu
from jax.experimental.pallas import tpu_sc as plsc

_B = 4096
_F = 26
_V = 100000
_D = 64
_NF = _F + 1
_NUM_INTER = (_NF * (_NF - 1)) // 2
_B_BLK = 256
_NW = 32
_CH = 128
_VB = 6272
_NVB = 4
_Q = _VB * _NVB


def _conv_body(a_ref, b_ref, c_ref, d_ref, eye_ref, out_ref):
  c00 = (((0,), (0,)), ((), ()))
  eye = eye_ref[...]
  f32 = jnp.float32
  half = jnp.uint32(0x8000)
  himask = jnp.uint32(0xFFFF0000)
  us = []
  for r in (a_ref, b_ref, c_ref, d_ref):
    t = lax.dot_general(r[...], eye, c00, preferred_element_type=f32)
    us.append(lax.bitcast_convert_type(t, jnp.uint32))
  w01 = ((us[1] + half) & himask) | lax.shift_right_logical(
      us[0] + half, jnp.uint32(16))
  w23 = ((us[3] + half) & himask) | lax.shift_right_logical(
      us[2] + half, jnp.uint32(16))
  out_ref[...] = lax.bitcast_convert_type(
      jnp.concatenate([w01, w23], axis=1), jnp.int32)


def _tc_convert(table2):
  return pl.pallas_call(
      _conv_body,
      grid=(_F, _NVB),
      in_specs=[
          pl.BlockSpec((_D, _VB), lambda f, v: (f, v)),
          pl.BlockSpec((_D, _VB), lambda f, v: (f, _NVB + v)),
          pl.BlockSpec((_D, _VB), lambda f, v: (f, 2 * _NVB + v)),
          pl.BlockSpec((_D, _VB), lambda f, v: (f, 3 * _NVB + v)),
          pl.BlockSpec((_D, _D), lambda f, v: (0, 0)),
      ],
      out_specs=pl.BlockSpec((_VB, 2 * _D), lambda f, v: (f * _NVB + v, 0)),
      out_shape=jax.ShapeDtypeStruct((_F * _Q, 2 * _D), jnp.int32),
  )(table2, table2, table2, table2, jnp.eye(_D, dtype=jnp.float32))


def _sc_gather(table, pair_idx):
  R = pair_idx.shape[0]
  r_per_w = R // _NW
  n_ch = r_per_w // _CH
  mesh = plsc.VectorSubcoreMesh(core_axis_name="c", subcore_axis_name="s")

  @functools.partial(
      pl.kernel,
      mesh=mesh,
      out_type=jax.ShapeDtypeStruct((R, 2 * _D), jnp.int32),
      scratch_types=[
          pltpu.VMEM((_CH,), jnp.int32),
          pltpu.VMEM((_CH,), jnp.int32),
          pltpu.VMEM((_CH, 2 * _D), jnp.int32),
          pltpu.VMEM((_CH, 2 * _D), jnp.int32),
          pltpu.SemaphoreType.DMA,
          pltpu.SemaphoreType.DMA,
          pltpu.SemaphoreType.DMA,
      ],
  )
  def gather_k(table_hbm, idx_hbm, out_hbm, idx0, idx1, rows0, rows1,
               isem, gsem, osem):
    wid = lax.axis_index("s") * 2 + lax.axis_index("c")
    base = wid * r_per_w
    idx = (idx0, idx1)
    rows = (rows0, rows1)

    ic = [None, None]
    oc = [None, None]
    ic[0] = pltpu.async_copy(idx_hbm.at[pl.ds(base, _CH)], idx0, isem)
    for c in range(n_ch):
      p = c % 2
      b0 = base + c * _CH
      ic[p].wait()
      if oc[p] is not None:
        oc[p].wait()
      g = pltpu.async_copy(table_hbm.at[idx[p]], rows[p], gsem)
      if c + 1 < n_ch:
        ic[1 - p] = pltpu.async_copy(
            idx_hbm.at[pl.ds(b0 + _CH, _CH)], idx[1 - p], isem)
      g.wait()
      oc[p] = pltpu.async_copy(rows[p], out_hbm.at[pl.ds(b0, _CH)], osem)
    for cp in oc:
      if cp is not None:
        cp.wait()

  return gather_k(table, pair_idx)


def _tc_body(emb_ref, par_ref, xdt_ref, bw0, bb0, bw1, bb1, bw2, bb2,
             tw0p, tb0, tw1, tb1, tw2, tb2, out_ref, sel_ref, et_ref, tt_ref):
  c00 = (((0,), (0,)), ((), ()))
  f32 = jnp.float32
  h = jnp.maximum(
      lax.dot_general(bw0[...], xdt_ref[...], c00, preferred_element_type=f32)
      + bb0[...], 0.0)
  h = jnp.maximum(
      lax.dot_general(bw1[...], h, c00, preferred_element_type=f32)
      + bb1[...], 0.0)
  dot_t = (lax.dot_general(bw2[...], h, c00, preferred_element_type=f32)
           + bb2[...])

  himask = jnp.uint32(0xFFFF0000)
  for f in range(_F):
    m32 = lax.bitcast_convert_type(emb_ref[f], jnp.uint32)
    q = par_ref[f].reshape(_B_BLK, 1)
    b0 = (q & 1) == 1
    b1 = q >= 2
    mq = jnp.where(b1, m32[:, _D:], m32[:, :_D])
    fq = lax.bitcast_convert_type(
        jnp.where(b0, mq & himask, lax.shift_left(mq, jnp.uint32(16))), f32)
    sel_ref[:, f * _D:(f + 1) * _D] = fq
  et_ref[...] = jnp.transpose(sel_ref[...])

  tt_ref[_NUM_INTER + 1:_NUM_INTER + 1 + _D, :] = dot_t
  tt_ref[_NUM_INTER:_NUM_INTER + 1, :] = jnp.zeros((1, _B_BLK), f32)

  p = 0
  for i in range(_NF):
    ai = et_ref[i * _D:(i + 1) * _D, :] if i < _F else dot_t
    for j in range(i + 1, _NF):
      aj = et_ref[j * _D:(j + 1) * _D, :] if j < _F else dot_t
      s = jnp.sum(ai * aj, axis=0, keepdims=True)
      tt_ref[p:p + 1, :] = s
      p += 1

  tt = tt_ref[...]
  h2 = jnp.maximum(
      lax.dot_general(tt, tw0p[...], c00, preferred_element_type=f32)
      + tb0[...], 0.0)
  h3 = jnp.maximum(
      jnp.dot(h2, tw1[...], preferred_element_type=f32) + tb1[...], 0.0)
  out_ref[...] = jnp.dot(h3, tw2[...], preferred_element_type=f32) + tb2[...]


def _tc_dense(emb3, par, xdt, bw0, bb0, bw1, bb1, bw2, bb2,
              tw0p, tb0, tw1, tb1, tw2, tb2):
  n_blk = _B // _B_BLK
  full = lambda shape: pl.BlockSpec(shape, lambda i: (0,) * len(shape))
  return pl.pallas_call(
      _tc_body,
      grid=(n_blk,),
      in_specs=[
          pl.BlockSpec((_F, _B_BLK, 2 * _D), lambda i: (0, i, 0)),
          pl.BlockSpec((_F, _B_BLK), lambda i: (0, i)),
          pl.BlockSpec((13, _B_BLK), lambda i: (0, i)),
          full(bw0.shape), full(bb0.shape), full(bw1.shape), full(bb1.shape),
          full(bw2.shape), full(bb2.shape), full(tw0p.shape), full(tb0.shape),
          full(tw1.shape), full(tb1.shape), full(tw2.shape), full(tb2.shape),
      ],
      out_specs=pl.BlockSpec((_B_BLK, 1), lambda i: (i, 0)),
      out_shape=jax.ShapeDtypeStruct((_B, 1), jnp.float32),
      scratch_shapes=[
          pltpu.VMEM((_B_BLK, _F * _D), jnp.float32),
          pltpu.VMEM((_F * _D, _B_BLK), jnp.float32),
          pltpu.VMEM((_NUM_INTER + 1 + _D, _B_BLK), jnp.float32),
      ],
  )(emb3, par, xdt, bw0, bb0, bw1, bb1, bw2, bb2,
    tw0p, tb0, tw1, tb1, tw2, tb2)


def kernel(x_sparse, x_dense, emb, bw0, bb0, bw1, bb1, bw2, bb2,
           tw0, tb0, tw1, tb1, tw2, tb2):
  table2 = jnp.transpose(emb, (0, 2, 1)).reshape(_F * _D, _V)
  table = _tc_convert(table2)

  xs_t = x_sparse.T
  par = xs_t // _Q
  pair_idx = (jnp.arange(_F, dtype=jnp.int32)[:, None] * _Q
              + xs_t - par * _Q).reshape(-1)
  embeds = _sc_gather(table, pair_idx)
  emb3 = embeds.reshape(_F, _B, 2 * _D)

  tw0p = jnp.concatenate(
      [tw0[:_NUM_INTER], jnp.zeros((1, tw0.shape[1]), tw0.dtype),
       tw0[_NUM_INTER:]], axis=0)
  out = _tc_dense(
      emb3, par, x_dense.T,
      bw0, bb0.reshape(-1, 1), bw1, bb1.reshape(-1, 1), bw2, bb2.reshape(-1, 1),
      tw0p, tb0.reshape(1, -1), tw1, tb1.reshape(1, -1), tw2, tb2.reshape(1, -1))
  return out


# speedup vs baseline: 2.5012x; 1.1768x over previous
import functools
import numpy as np
import jax
import jax.numpy as jnp
from jax import lax
from jax.experimental import pallas as pl
from jax.experimental.pallas import tpu as pltpu
from jax.experimental.pallas import tpu_sc as plsc

_B = 4096
_F = 26
_V = 100000
_D = 64
_NF = _F + 1
_NUM_INTER = (_NF * (_NF - 1)) // 2
_B_BLK = 256
_NW = 32
_CH = 128
_VB = 6272
_NVB = 4
_Q = _VB * _NVB


def _conv_body(a_ref, b_ref, c_ref, d_ref, eye_ref, out_ref):
  c00 = (((0,), (0,)), ((), ()))
  eye = eye_ref[...]
  f32 = jnp.float32
  half = jnp.uint32(0x8000)
  himask = jnp.uint32(0xFFFF0000)
  us = []
  bf16 = jnp.bfloat16
  eye_b = eye.astype(bf16)
  for r in (a_ref, b_ref, c_ref, d_ref):
    t = lax.dot_general(r[...].astype(bf16), eye_b, c00,
                        preferred_element_type=f32)
    us.append(lax.bitcast_convert_type(t, jnp.uint32))
  w01 = ((us[1] + half) & himask) | lax.shift_right_logical(
      us[0] + half, jnp.uint32(16))
  w23 = ((us[3] + half) & himask) | lax.shift_right_logical(
      us[2] + half, jnp.uint32(16))
  out_ref[...] = lax.bitcast_convert_type(
      jnp.concatenate([w01, w23], axis=1), jnp.int32)


def _tc_convert(table2):
  return pl.pallas_call(
      _conv_body,
      grid=(_F, _NVB),
      in_specs=[
          pl.BlockSpec((_D, _VB), lambda f, v: (f, v)),
          pl.BlockSpec((_D, _VB), lambda f, v: (f, _NVB + v)),
          pl.BlockSpec((_D, _VB), lambda f, v: (f, 2 * _NVB + v)),
          pl.BlockSpec((_D, _VB), lambda f, v: (f, 3 * _NVB + v)),
          pl.BlockSpec((_D, _D), lambda f, v: (0, 0)),
      ],
      out_specs=pl.BlockSpec((_VB, 2 * _D), lambda f, v: (f * _NVB + v, 0)),
      out_shape=jax.ShapeDtypeStruct((_F * _Q, 2 * _D), jnp.int32),
  )(table2, table2, table2, table2, jnp.eye(_D, dtype=jnp.float32))


def _sc_gather(table, pair_idx):
  R = pair_idx.shape[0]
  r_per_w = R // _NW
  n_ch = r_per_w // _CH
  mesh = plsc.VectorSubcoreMesh(core_axis_name="c", subcore_axis_name="s")

  @functools.partial(
      pl.kernel,
      mesh=mesh,
      out_type=jax.ShapeDtypeStruct((R, 2 * _D), jnp.int32),
      scratch_types=[
          pltpu.VMEM((_CH,), jnp.int32),
          pltpu.VMEM((_CH,), jnp.int32),
          pltpu.VMEM((_CH, 2 * _D), jnp.int32),
          pltpu.VMEM((_CH, 2 * _D), jnp.int32),
          pltpu.SemaphoreType.DMA,
          pltpu.SemaphoreType.DMA,
          pltpu.SemaphoreType.DMA,
      ],
  )
  def gather_k(table_hbm, idx_hbm, out_hbm, idx0, idx1, rows0, rows1,
               isem, gsem, osem):
    wid = lax.axis_index("s") * 2 + lax.axis_index("c")
    base = wid * r_per_w
    idx = (idx0, idx1)
    rows = (rows0, rows1)

    ic = [None, None]
    oc = [None, None]
    ic[0] = pltpu.async_copy(idx_hbm.at[pl.ds(base, _CH)], idx0, isem)
    for c in range(n_ch):
      p = c % 2
      b0 = base + c * _CH
      ic[p].wait()
      if oc[p] is not None:
        oc[p].wait()
      g = pltpu.async_copy(table_hbm.at[idx[p]], rows[p], gsem)
      if c + 1 < n_ch:
        ic[1 - p] = pltpu.async_copy(
            idx_hbm.at[pl.ds(b0 + _CH, _CH)], idx[1 - p], isem)
      g.wait()
      oc[p] = pltpu.async_copy(rows[p], out_hbm.at[pl.ds(b0, _CH)], osem)
    for cp in oc:
      if cp is not None:
        cp.wait()

  return gather_k(table, pair_idx)


def _tc_body(emb_ref, par_ref, xdt_ref, bw0, bb0, bw1, bb1, bw2, bb2,
             tw0p, tb0, tw1, tb1, tw2, tb2, out_ref, sel_ref, et_ref, tt_ref):
  c00 = (((0,), (0,)), ((), ()))
  f32 = jnp.float32
  h = jnp.maximum(
      lax.dot_general(bw0[...], xdt_ref[...], c00, preferred_element_type=f32)
      + bb0[...], 0.0)
  h = jnp.maximum(
      lax.dot_general(bw1[...], h, c00, preferred_element_type=f32)
      + bb1[...], 0.0)
  dot_t = (lax.dot_general(bw2[...], h, c00, preferred_element_type=f32)
           + bb2[...])

  himask = jnp.uint32(0xFFFF0000)
  for f in range(_F):
    m32 = lax.bitcast_convert_type(emb_ref[f], jnp.uint32)
    q = par_ref[f].reshape(_B_BLK, 1)
    b0 = (q & 1) == 1
    b1 = q >= 2
    mq = jnp.where(b1, m32[:, _D:], m32[:, :_D])
    fq = lax.bitcast_convert_type(
        jnp.where(b0, mq & himask, lax.shift_left(mq, jnp.uint32(16))), f32)
    sel_ref[:, f * _D:(f + 1) * _D] = fq
  et_ref[...] = jnp.transpose(sel_ref[...])

  tt_ref[_NUM_INTER + 1:_NUM_INTER + 1 + _D, :] = dot_t
  tt_ref[_NUM_INTER:_NUM_INTER + 1, :] = jnp.zeros((1, _B_BLK), f32)

  p = 0
  for i in range(_NF):
    ai = et_ref[i * _D:(i + 1) * _D, :] if i < _F else dot_t
    for j in range(i + 1, _NF):
      aj = et_ref[j * _D:(j + 1) * _D, :] if j < _F else dot_t
      s = jnp.sum(ai * aj, axis=0, keepdims=True)
      tt_ref[p:p + 1, :] = s
      p += 1

  tt = tt_ref[...]
  h2 = jnp.maximum(
      lax.dot_general(tt, tw0p[...], c00, preferred_element_type=f32)
      + tb0[...], 0.0)
  h3 = jnp.maximum(
      jnp.dot(h2, tw1[...], preferred_element_type=f32) + tb1[...], 0.0)
  out_ref[...] = jnp.dot(h3, tw2[...], preferred_element_type=f32) + tb2[...]


def _tc_dense(emb3, par, xdt, bw0, bb0, bw1, bb1, bw2, bb2,
              tw0p, tb0, tw1, tb1, tw2, tb2):
  n_blk = _B // _B_BLK
  full = lambda shape: pl.BlockSpec(shape, lambda i: (0,) * len(shape))
  return pl.pallas_call(
      _tc_body,
      grid=(n_blk,),
      in_specs=[
          pl.BlockSpec((_F, _B_BLK, 2 * _D), lambda i: (0, i, 0)),
          pl.BlockSpec((_F, _B_BLK), lambda i: (0, i)),
          pl.BlockSpec((13, _B_BLK), lambda i: (0, i)),
          full(bw0.shape), full(bb0.shape), full(bw1.shape), full(bb1.shape),
          full(bw2.shape), full(bb2.shape), full(tw0p.shape), full(tb0.shape),
          full(tw1.shape), full(tb1.shape), full(tw2.shape), full(tb2.shape),
      ],
      out_specs=pl.BlockSpec((_B_BLK, 1), lambda i: (i, 0)),
      out_shape=jax.ShapeDtypeStruct((_B, 1), jnp.float32),
      scratch_shapes=[
          pltpu.VMEM((_B_BLK, _F * _D), jnp.float32),
          pltpu.VMEM((_F * _D, _B_BLK), jnp.float32),
          pltpu.VMEM((_NUM_INTER + 1 + _D, _B_BLK), jnp.float32),
      ],
  )(emb3, par, xdt, bw0, bb0, bw1, bb1, bw2, bb2,
    tw0p, tb0, tw1, tb1, tw2, tb2)


def kernel(x_sparse, x_dense, emb, bw0, bb0, bw1, bb1, bw2, bb2,
           tw0, tb0, tw1, tb1, tw2, tb2):
  table2 = jnp.transpose(emb, (0, 2, 1)).reshape(_F * _D, _V)
  table = _tc_convert(table2)

  xs_t = x_sparse.T
  par = xs_t // _Q
  pair_idx = (jnp.arange(_F, dtype=jnp.int32)[:, None] * _Q
              + xs_t - par * _Q).reshape(-1)
  embeds = _sc_gather(table, pair_idx)
  emb3 = embeds.reshape(_F, _B, 2 * _D)

  tw0p = jnp.concatenate(
      [tw0[:_NUM_INTER], jnp.zeros((1, tw0.shape[1]), tw0.dtype),
       tw0[_NUM_INTER:]], axis=0)
  out = _tc_dense(
      emb3, par, x_dense.T,
      bw0, bb0.reshape(-1, 1), bw1, bb1.reshape(-1, 1), bw2, bb2.reshape(-1, 1),
      tw0p, tb0.reshape(1, -1), tw1, tb1.reshape(1, -1), tw2, tb2.reshape(1, -1))
  return out
